# baseline (speedup 1.0000x reference)
import functools

import jax
import jax.numpy as jnp
from jax import lax
from jax.experimental import pallas as pl
from jax.experimental.pallas import tpu as pltpu
from jax.experimental.pallas import tpu_sc as plsc

NC, NS = 2, 16
NW = NC * NS
LANES = 16
PB = 128


@functools.cache
def _tpe_kernel(B, L, D):
    NPB = L // PB
    NBH = NW // NPB
    BH = B // NBH
    assert NPB * PB == L and NBH * BH == B and D % LANES == 0
    mesh = plsc.VectorSubcoreMesh(core_axis_name="c", subcore_axis_name="s")

    @functools.partial(
        pl.kernel,
        out_type=jax.ShapeDtypeStruct((B * L, D), jnp.float32),
        mesh=mesh,
        scratch_types=[
            pltpu.VMEM((BH, PB), jnp.int32),
            pltpu.VMEM((PB,), jnp.int32),
            pltpu.VMEM((PB,), jnp.int32),
            pltpu.VMEM((PB, D), jnp.float32),
            pltpu.VMEM((PB, 2 * D), jnp.float32),
            pltpu.VMEM((PB, D), jnp.float32),
            pltpu.SemaphoreType.DMA,
        ],
        compiler_params=pltpu.CompilerParams(needs_layout_passes=False),
    )
    def k(x_hbm, tok_hbm, pos_hbm, out_hbm, idx_v, row_v, half_v, pos_v,
          rows_v, out_v, sem):
        wid = lax.axis_index("s") * NC + lax.axis_index("c")
        pb = lax.rem(wid, NPB)
        bh = wid // NPB
        pbase = pb * PB
        b0 = bh * BH
        pltpu.sync_copy(pos_hbm.at[pl.ds(pbase, PB)], pos_v)
        pltpu.sync_copy(x_hbm.at[pl.ds(b0, BH), pl.ds(pbase, PB)], idx_v)

        def per_batch(b, carry):
            def split_ids(j, c):
                sl = pl.ds(j * LANES, LANES)
                t = idx_v[b, sl]
                row_v[sl] = lax.shift_right_logical(t, 1)
                half_v[sl] = lax.bitwise_and(t, 1)
                return c

            lax.fori_loop(0, PB // LANES, split_ids, 0)
            pltpu.async_copy(tok_hbm.at[row_v], rows_v, sem).wait()

            def extract_row(r, c):
                rsplat = jnp.full((LANES,), r, jnp.int32)
                hsplat = plsc.load_gather(half_v, [rsplat])
                lane = lax.iota(jnp.int32, LANES)
                for j in range(D // LANES):
                    col = hsplat * D + j * LANES + lane
                    vals = plsc.load_gather(rows_v, [rsplat, col])
                    out_v[r, pl.ds(j * LANES, LANES)] = (
                        vals + pos_v[r, pl.ds(j * LANES, LANES)])
                return c

            lax.fori_loop(0, PB, extract_row, 0)
            pltpu.sync_copy(out_v, out_hbm.at[pl.ds((b0 + b) * L + pbase, PB)])
            return carry

        lax.fori_loop(0, BH, per_batch, 0)

    return k


def kernel(x, token_table, pos_table):
    B, L = x.shape
    V, D = token_table.shape
    tok2 = token_table.reshape(V // 2, 2 * D)
    flat = _tpe_kernel(B, L, D)(x.astype(jnp.int32), tok2, pos_table)
    return flat.reshape(B, L, D)


# speedup vs baseline: 1.2535x; 1.2535x over previous
import functools

import jax
import jax.numpy as jnp
from jax import lax
from jax.experimental import pallas as pl
from jax.experimental.pallas import tpu as pltpu
from jax.experimental.pallas import tpu_sc as plsc

NC, NS = 2, 16
NW = NC * NS
LANES = 16
PB = 128


@functools.cache
def _tpe_kernel(B, L, D):
    NPB = L // PB
    NBH = NW // NPB
    BH = B // NBH
    assert NPB * PB == L and NBH * BH == B and D % LANES == 0
    mesh = plsc.VectorSubcoreMesh(core_axis_name="c", subcore_axis_name="s")

    @functools.partial(
        pl.kernel,
        out_type=jax.ShapeDtypeStruct((B, D, L), jnp.float32),
        mesh=mesh,
        scratch_types=[
            pltpu.VMEM((BH, PB), jnp.int32),
            pltpu.VMEM((PB,), jnp.int32),
            pltpu.VMEM((D, PB), jnp.float32),
            pltpu.VMEM((PB // 2, 8, D), jnp.float32),
            pltpu.VMEM((D, PB), jnp.float32),
            pltpu.SemaphoreType.DMA,
            pltpu.SemaphoreType.DMA,
        ],
        compiler_params=pltpu.CompilerParams(needs_layout_passes=False),
    )
    def k(x_hbm, tok_hbm, posT_hbm, out_hbm, idx_v, t8_v, posT_v, blk_v,
          out_v, gsem, osem):
        wid = lax.axis_index("s") * NC + lax.axis_index("c")
        pb = lax.rem(wid, NPB)
        bh = wid // NPB
        pbase = pb * PB
        b0 = bh * BH
        pltpu.sync_copy(posT_hbm.at[:, pl.ds(pbase, PB)], posT_v)
        pltpu.sync_copy(x_hbm.at[pl.ds(b0, BH), pl.ds(pbase, PB)], idx_v)
        lane = lax.iota(jnp.int32, LANES)

        HALF = PB // 2

        def per_batch(b, carry):
            for h in range(2):
                def fetch_chunk(i, c, h=h):
                    tvec = idx_v[b, pl.ds(h * HALF + i * LANES, LANES)]
                    t8_v[pl.ds(h * HALF + i * LANES, LANES)] = lax.rem(tvec, 8)
                    copies = []
                    for kk in range(LANES):
                        t = tvec[kk]
                        copies.append(pltpu.async_copy(
                            tok_hbm.at[t // 8],
                            blk_v.at[i * LANES + kk], gsem))
                    for cp in copies:
                        cp.wait()
                    return c

                lax.fori_loop(0, HALF // LANES, fetch_chunk, 0)

                def emit_row(d, c, h=h):
                    dsplat = jnp.full((LANES,), d, jnp.int32)
                    for m in range(HALF // LANES):
                        sl = pl.ds(h * HALF + m * LANES, LANES)
                        rvec = lane + (m * LANES)
                        t8c = t8_v[sl]
                        vals = plsc.load_gather(blk_v, [rvec, t8c, dsplat])
                        out_v[d, sl] = vals + posT_v[d, sl]
                    return c

                lax.fori_loop(0, D, emit_row, 0)
            pltpu.sync_copy(out_v, out_hbm.at[b0 + b, :, pl.ds(pbase, PB)])
            return carry

        lax.fori_loop(0, BH, per_batch, 0)

    return k


def kernel(x, token_table, pos_table):
    B, L = x.shape
    V, D = token_table.shape
    tok3 = token_table.reshape(V // 8, 8, D)
    out_t = _tpe_kernel(B, L, D)(x.astype(jnp.int32), tok3, pos_table.T)
    return out_t.transpose(0, 2, 1)


# speedup vs baseline: 1.8143x; 1.4474x over previous
import functools

import jax
import jax.numpy as jnp
from jax import lax
from jax.experimental import pallas as pl
from jax.experimental.pallas import tpu as pltpu
from jax.experimental.pallas import tpu_sc as plsc

NC, NS = 2, 16
NW = NC * NS
LANES = 16
PB = 128


@functools.cache
def _tpe_kernel(B, L, D):
    NPB = L // PB
    NBH = NW // NPB
    BH = B // NBH
    assert NPB * PB == L and NBH * BH == B and D % LANES == 0
    mesh = plsc.VectorSubcoreMesh(core_axis_name="c", subcore_axis_name="s")
    NCH = PB // LANES

    @functools.partial(
        pl.kernel,
        out_type=jax.ShapeDtypeStruct((B * L, D), jnp.float32),
        mesh=mesh,
        scratch_types=[
            pltpu.VMEM((BH, PB), jnp.int32),
            pltpu.VMEM((2 * LANES, 8, D), jnp.float32),
            pltpu.VMEM((PB, D), jnp.float32),
            pltpu.VMEM((PB, D), jnp.float32),
            pltpu.SemaphoreType.DMA,
        ],
        compiler_params=pltpu.CompilerParams(needs_layout_passes=False),
    )
    def k(x_hbm, tok_hbm, pos_hbm, out_hbm, idx_v, blk_v, pos_v, out_v, gsem):
        wid = lax.axis_index("s") * NC + lax.axis_index("c")
        pb = lax.rem(wid, NPB)
        bh = wid // NPB
        pbase = pb * PB
        b0 = bh * BH
        pltpu.sync_copy(pos_hbm.at[pl.ds(pbase, PB)], pos_v)
        pltpu.sync_copy(x_hbm.at[pl.ds(b0, BH), pl.ds(pbase, PB)], idx_v)

        def fire_chunk(b, i):
            slot = (i % 2) * LANES
            tvec = idx_v[b, pl.ds(i * LANES, LANES)]
            copies = []
            for kk in range(LANES):
                copies.append(pltpu.async_copy(
                    tok_hbm.at[tvec[kk] // 8], blk_v.at[slot + kk], gsem))
            return copies

        def extract_chunk(b, i):
            slot = (i % 2) * LANES
            t8vec = lax.rem(idx_v[b, pl.ds(i * LANES, LANES)], 8)
            for kk in range(LANES):
                r = i * LANES + kk
                t8 = t8vec[kk]
                for j in range(D // LANES):
                    sl = pl.ds(j * LANES, LANES)
                    out_v[r, sl] = blk_v[slot + kk, t8, sl] + pos_v[r, sl]

        def per_batch(b, carry):
            pending = fire_chunk(b, 0)
            for i in range(NCH):
                nxt = fire_chunk(b, i + 1) if i + 1 < NCH else []
                for cp in pending:
                    cp.wait()
                extract_chunk(b, i)
                pending = nxt
            pltpu.sync_copy(out_v, out_hbm.at[pl.ds((b0 + b) * L + pbase, PB)])
            return carry

        lax.fori_loop(0, BH, per_batch, 0)

    return k


def kernel(x, token_table, pos_table):
    B, L = x.shape
    V, D = token_table.shape
    tok3 = token_table.reshape(V // 8, 8, D)
    flat = _tpe_kernel(B, L, D)(x.astype(jnp.int32), tok3, pos_table)
    return flat.reshape(B, L, D)


# speedup vs baseline: 1.9042x; 1.0495x over previous
import functools

import jax
import jax.numpy as jnp
from jax import lax
from jax.experimental import pallas as pl
from jax.experimental.pallas import tpu as pltpu
from jax.experimental.pallas import tpu_sc as plsc

NC, NS = 2, 16
NW = NC * NS
LANES = 16
PB = 128


@functools.cache
def _tpe_kernel(B, L, D):
    NPB = L // PB
    NBH = NW // NPB
    BH = B // NBH
    assert NPB * PB == L and NBH * BH == B and D % LANES == 0
    mesh = plsc.VectorSubcoreMesh(core_axis_name="c", subcore_axis_name="s")
    NCH = PB // LANES

    @functools.partial(
        pl.kernel,
        out_type=jax.ShapeDtypeStruct((B * L, D), jnp.float32),
        mesh=mesh,
        scratch_types=[
            pltpu.VMEM((BH, PB), jnp.int32),
            pltpu.VMEM((4 * LANES, 8, D), jnp.float32),
            pltpu.VMEM((PB, D), jnp.float32),
            pltpu.VMEM((PB, D), jnp.float32),
            pltpu.SemaphoreType.DMA,
        ],
        compiler_params=pltpu.CompilerParams(needs_layout_passes=False),
    )
    def k(x_hbm, tok_hbm, pos_hbm, out_hbm, idx_v, blk_v, pos_v, out_v, gsem):
        wid = lax.axis_index("s") * NC + lax.axis_index("c")
        pb = lax.rem(wid, NPB)
        bh = wid // NPB
        pbase = pb * PB
        b0 = bh * BH
        pltpu.sync_copy(pos_hbm.at[pl.ds(pbase, PB)], pos_v)
        pltpu.sync_copy(x_hbm.at[pl.ds(b0, BH), pl.ds(pbase, PB)], idx_v)

        def fire_chunk(b, i):
            slot = (i % 4) * LANES
            tvec = idx_v[b, pl.ds(i * LANES, LANES)]
            copies = []
            for kk in range(LANES):
                copies.append(pltpu.async_copy(
                    tok_hbm.at[tvec[kk] // 8], blk_v.at[slot + kk], gsem))
            return copies

        def extract_chunk(b, i):
            slot = (i % 4) * LANES
            t8vec = lax.rem(idx_v[b, pl.ds(i * LANES, LANES)], 8)
            for kk in range(LANES):
                r = i * LANES + kk
                t8 = t8vec[kk]
                for j in range(D // LANES):
                    sl = pl.ds(j * LANES, LANES)
                    out_v[r, sl] = blk_v[slot + kk, t8, sl] + pos_v[r, sl]

        AHEAD = 3

        def per_batch(b, carry):
            pending = [fire_chunk(b, i) for i in range(AHEAD)]
            for i in range(NCH):
                if i + AHEAD < NCH:
                    pending.append(fire_chunk(b, i + AHEAD))
                for cp in pending.pop(0):
                    cp.wait()
                extract_chunk(b, i)
            pltpu.sync_copy(out_v, out_hbm.at[pl.ds((b0 + b) * L + pbase, PB)])
            return carry

        lax.fori_loop(0, BH, per_batch, 0)

    return k


def kernel(x, token_table, pos_table):
    B, L = x.shape
    V, D = token_table.shape
    tok3 = token_table.reshape(V // 8, 8, D)
    flat = _tpe_kernel(B, L, D)(x.astype(jnp.int32), tok3, pos_table)
    return flat.reshape(B, L, D)


# speedup vs baseline: 2.0264x; 1.0642x over previous
import functools

import jax
import jax.numpy as jnp
from jax import lax
from jax.experimental import pallas as pl
from jax.experimental.pallas import tpu as pltpu
from jax.experimental.pallas import tpu_sc as plsc

NC, NS = 2, 16
NW = NC * NS
LANES = 16
PB = 128


@functools.cache
def _tpe_kernel(B, L, D):
    NPB = L // PB
    NBH = NW // NPB
    BH = B // NBH
    assert NPB * PB == L and NBH * BH == B and D % LANES == 0
    mesh = plsc.VectorSubcoreMesh(core_axis_name="c", subcore_axis_name="s")
    NCH = PB // LANES

    @functools.partial(
        pl.kernel,
        out_type=jax.ShapeDtypeStruct((B * L, D), jnp.float32),
        mesh=mesh,
        scratch_types=[
            pltpu.VMEM((BH, PB), jnp.int32),
            pltpu.VMEM((4 * LANES, D), jnp.float32),
            pltpu.VMEM((PB, D), jnp.float32),
            pltpu.VMEM((PB, D), jnp.float32),
            pltpu.SemaphoreType.DMA,
        ],
        compiler_params=pltpu.CompilerParams(needs_layout_passes=False),
    )
    def k(x_hbm, tok_hbm, pos_hbm, out_hbm, idx_v, blk_v, pos_v, out_v, gsem):
        wid = lax.axis_index("s") * NC + lax.axis_index("c")
        pb = lax.rem(wid, NPB)
        bh = wid // NPB
        pbase = pb * PB
        b0 = bh * BH
        pltpu.sync_copy(pos_hbm.at[pl.ds(pbase, PB)], pos_v)
        pltpu.sync_copy(x_hbm.at[pl.ds(b0, BH), pl.ds(pbase, PB)], idx_v)

        def fire_chunk(b, i):
            slot = (i % 4) * LANES
            tvec = idx_v[b, pl.ds(i * LANES, LANES)]
            copies = []
            for kk in range(LANES):
                t = tvec[kk]
                copies.append(pltpu.async_copy(
                    tok_hbm.at[t // 8, lax.rem(t, 8)],
                    blk_v.at[slot + kk], gsem))
            return copies

        def extract_chunk(b, i):
            slot = (i % 4) * LANES
            for kk in range(LANES):
                r = i * LANES + kk
                for j in range(D // LANES):
                    sl = pl.ds(j * LANES, LANES)
                    out_v[r, sl] = blk_v[slot + kk, sl] + pos_v[r, sl]

        AHEAD = 3

        def per_batch(b, carry):
            pending = [fire_chunk(b, i) for i in range(AHEAD)]
            for i in range(NCH):
                if i + AHEAD < NCH:
                    pending.append(fire_chunk(b, i + AHEAD))
                for cp in pending.pop(0):
                    cp.wait()
                extract_chunk(b, i)
            pltpu.sync_copy(out_v, out_hbm.at[pl.ds((b0 + b) * L + pbase, PB)])
            return carry

        lax.fori_loop(0, BH, per_batch, 0)

    return k


def kernel(x, token_table, pos_table):
    B, L = x.shape
    V, D = token_table.shape
    tok3 = token_table.reshape(V // 8, 8, D)
    flat = _tpe_kernel(B, L, D)(x.astype(jnp.int32), tok3, pos_table)
    return flat.reshape(B, L, D)


# speedup vs baseline: 2.0958x; 1.0343x over previous
import functools

import jax
import jax.numpy as jnp
from jax import lax
from jax.experimental import pallas as pl
from jax.experimental.pallas import tpu as pltpu
from jax.experimental.pallas import tpu_sc as plsc

NC, NS = 2, 16
NW = NC * NS
LANES = 16
PB = 128


@functools.cache
def _tpe_kernel(B, L, D):
    NPB = L // PB
    NBH = NW // NPB
    BH = B // NBH
    assert NPB * PB == L and NBH * BH == B and D % LANES == 0
    mesh = plsc.VectorSubcoreMesh(core_axis_name="c", subcore_axis_name="s")
    NCH = PB // LANES

    @functools.partial(
        pl.kernel,
        out_type=jax.ShapeDtypeStruct((B * L, D), jnp.float32),
        mesh=mesh,
        scratch_types=[
            pltpu.VMEM((BH, PB), jnp.int32),
            pltpu.VMEM((8 * LANES, D), jnp.float32),
            pltpu.VMEM((PB, D), jnp.float32),
            pltpu.VMEM((PB, D), jnp.float32),
        ] + [pltpu.SemaphoreType.DMA] * (PB // LANES),
        compiler_params=pltpu.CompilerParams(needs_layout_passes=False),
    )
    def k(x_hbm, tok_hbm, pos_hbm, out_hbm, idx_v, blk_v, pos_v, out_v,
          *gsems):
        wid = lax.axis_index("s") * NC + lax.axis_index("c")
        pb = lax.rem(wid, NPB)
        bh = wid // NPB
        pbase = pb * PB
        b0 = bh * BH
        pltpu.sync_copy(pos_hbm.at[pl.ds(pbase, PB)], pos_v)
        pltpu.sync_copy(x_hbm.at[pl.ds(b0, BH), pl.ds(pbase, PB)], idx_v)

        def fire_chunk(b, i):
            slot = (i % 8) * LANES
            tvec = idx_v[b, pl.ds(i * LANES, LANES)]
            for kk in range(LANES):
                t = tvec[kk]
                pltpu.async_copy(
                    tok_hbm.at[t // 8, lax.rem(t, 8)],
                    blk_v.at[slot + kk], gsems[i % 8])
            return slot

        def drain_chunk(i, slot):
            pltpu.make_async_copy(
                out_hbm.at[pl.ds(0, LANES)],
                blk_v.at[pl.ds(slot, LANES), :], gsems[i % 8]).wait()

        def extract_chunk(b, i):
            slot = (i % 8) * LANES
            for kk in range(LANES):
                r = i * LANES + kk
                for j in range(D // LANES):
                    sl = pl.ds(j * LANES, LANES)
                    out_v[r, sl] = blk_v[slot + kk, sl] + pos_v[r, sl]

        AHEAD = 6

        def per_batch(b, carry):
            pending = [fire_chunk(b, i) for i in range(AHEAD)]
            for i in range(NCH):
                if i + AHEAD < NCH:
                    pending.append(fire_chunk(b, i + AHEAD))
                drain_chunk(i, pending.pop(0))
                extract_chunk(b, i)
            pltpu.sync_copy(out_v, out_hbm.at[pl.ds((b0 + b) * L + pbase, PB)])
            return carry

        lax.fori_loop(0, BH, per_batch, 0)

    return k


def kernel(x, token_table, pos_table):
    B, L = x.shape
    V, D = token_table.shape
    tok3 = token_table.reshape(V // 8, 8, D)
    flat = _tpe_kernel(B, L, D)(x.astype(jnp.int32), tok3, pos_table)
    return flat.reshape(B, L, D)
